# baseline (speedup 1.0000x reference)
import functools

import jax
import jax.numpy as jnp
from jax import lax
from jax.experimental import pallas as pl
from jax.experimental.pallas import tpu as pltpu
from jax.experimental.pallas import tpu_sc as plsc

B = 128
A = 64
N1 = B * A + 1
MAXNB = 16
E1 = B * A * MAXNB + 1
AF = 256
BF = 256
H = 512
DEPTH = 6

E1P = 132096
N1P = 8832

NC = 2
NS = 16
NW = NC * NS
L = 16

@functools.lru_cache(maxsize=None)
def _sc_mesh():
    return plsc.VectorSubcoreMesh(core_axis_name="c", subcore_axis_name="s",
                                  num_cores=NC, num_subcores=NS)


_A_RW = N1P // NW
_A_CR = 4
_A_NB = 3
_A_CH = _A_RW // _A_CR
_A_GP = _A_CH // _A_NB


def _sc_gather_sum_body(msg_hbm, a2b_hbm, out_hbm, idx_v, rows_v, acc_v,
                        sg0, sg1, sg2, so0, so1, so2):
    wid = lax.axis_index("s") * NC + lax.axis_index("c")
    base = wid * _A_RW
    sgs = [sg0, sg1, sg2]
    sos = [so0, so1, so2]

    pltpu.sync_copy(a2b_hbm.at[pl.ds(base * MAXNB, _A_RW * MAXNB)], idx_v)

    _HC = _A_CR * MAXNB // 2

    def gather(ci, b):
        i0 = ci * _A_CR * MAXNB
        return (pltpu.make_async_copy(
                    msg_hbm.at[idx_v.at[pl.ds(i0, _HC)]],
                    rows_v.at[b, pl.ds(0, _HC)], sgs[b]),
                pltpu.make_async_copy(
                    msg_hbm.at[idx_v.at[pl.ds(i0 + _HC, _HC)]],
                    rows_v.at[b, pl.ds(_HC, _HC)], sgs[b]))

    for b in range(_A_NB):
        for cp in gather(b, b):
            cp.start()

    def grp(g, _):
        for b in range(_A_NB):
            ci = g * _A_NB + b
            row0 = base + ci * _A_CR
            for cp in gather(ci, b):
                cp.wait()

            @pl.when(g > 0)
            def _():
                pltpu.make_async_copy(acc_v.at[b],
                                      out_hbm.at[pl.ds(row0, _A_CR)],
                                      sos[b]).wait()

            def row(r, _):
                def col(c, _):
                    cc = c * L
                    acc = rows_v[b, r * MAXNB, pl.ds(cc, L)]
                    for j in range(1, MAXNB):
                        acc = acc + rows_v[b, r * MAXNB + j, pl.ds(cc, L)]
                    acc_v[b, r, pl.ds(cc, L)] = acc
                    return 0

                lax.fori_loop(0, H // L, col, 0)
                return 0

            lax.fori_loop(0, _A_CR, row, 0)
            pltpu.make_async_copy(acc_v.at[b],
                                  out_hbm.at[pl.ds(row0, _A_CR)],
                                  sos[b]).start()

            @pl.when(g < _A_GP - 1)
            def _():
                for cp in gather(ci + _A_NB, b):
                    cp.start()

        return 0

    lax.fori_loop(0, _A_GP, grp, 0)
    for b in range(_A_NB):
        row0 = base + (_A_CH - _A_NB + b) * _A_CR
        pltpu.make_async_copy(acc_v.at[b], out_hbm.at[pl.ds(row0, _A_CR)],
                              sos[b]).wait()


@functools.lru_cache(maxsize=None)
def _sc_gather_sum_kernel():
    return pl.kernel(
        _sc_gather_sum_body,
        mesh=_sc_mesh(),
        out_type=jax.ShapeDtypeStruct((N1P, H), jnp.float32),
        scratch_types=[
            pltpu.VMEM((_A_RW * MAXNB,), jnp.int32),
            pltpu.VMEM((_A_NB, _A_CR * MAXNB, H), jnp.float32),
            pltpu.VMEM((_A_NB, _A_CR, H), jnp.float32),
            pltpu.SemaphoreType.DMA,
            pltpu.SemaphoreType.DMA,
            pltpu.SemaphoreType.DMA,
            pltpu.SemaphoreType.DMA,
            pltpu.SemaphoreType.DMA,
            pltpu.SemaphoreType.DMA,
        ],
    )


def _sc_gather_sum(msg, a2b_f):
    return _sc_gather_sum_kernel()(msg, a2b_f)


_B_RW = E1P // NW
_B_CR = 16
_B_NB = 3
_B_CH = _B_RW // _B_CR
_B_GP = _B_CH // _B_NB


def _sc_bond_update_body(amsg_hbm, msg_hbm, b2a_hbm, b2revb_hbm, out_hbm,
                         idxa_v, idxb_v, rows_a, rows_b, out_v,
                         sa0, sa1, sa2, sb0, sb1, sb2, so0, so1, so2):
    wid = lax.axis_index("s") * NC + lax.axis_index("c")
    base = wid * _B_RW
    sas = [sa0, sa1, sa2]
    sbs = [sb0, sb1, sb2]
    sos = [so0, so1, so2]

    pltpu.sync_copy(b2a_hbm.at[pl.ds(base, _B_RW)], idxa_v)
    pltpu.sync_copy(b2revb_hbm.at[pl.ds(base, _B_RW)], idxb_v)

    _HC = _B_CR // 2

    def gathers(ci, b):
        i0 = ci * _B_CR
        lo, hi = pl.ds(i0, _HC), pl.ds(i0 + _HC, _HC)
        return (pltpu.make_async_copy(amsg_hbm.at[idxa_v.at[lo]],
                                      rows_a.at[b, pl.ds(0, _HC)], sas[b]),
                pltpu.make_async_copy(amsg_hbm.at[idxa_v.at[hi]],
                                      rows_a.at[b, pl.ds(_HC, _HC)], sas[b]),
                pltpu.make_async_copy(msg_hbm.at[idxb_v.at[lo]],
                                      rows_b.at[b, pl.ds(0, _HC)], sbs[b]),
                pltpu.make_async_copy(msg_hbm.at[idxb_v.at[hi]],
                                      rows_b.at[b, pl.ds(_HC, _HC)], sbs[b]))

    for b in range(_B_NB):
        for cp in gathers(b, b):
            cp.start()

    def grp(g, _):
        for b in range(_B_NB):
            ci = g * _B_NB + b
            row0 = base + ci * _B_CR
            for cp in gathers(ci, b):
                cp.wait()

            @pl.when(g > 0)
            def _():
                pltpu.make_async_copy(out_v.at[b],
                                      out_hbm.at[pl.ds(row0, _B_CR)],
                                      sos[b]).wait()

            def row(r, _):
                def col(c, _):
                    cc = c * L
                    out_v[b, r, pl.ds(cc, L)] = (
                        rows_a[b, r, pl.ds(cc, L)] - rows_b[b, r, pl.ds(cc, L)]
                    )
                    return 0

                lax.fori_loop(0, H // L, col, 0)
                return 0

            lax.fori_loop(0, _B_CR, row, 0)
            pltpu.make_async_copy(out_v.at[b],
                                  out_hbm.at[pl.ds(row0, _B_CR)],
                                  sos[b]).start()

            @pl.when(g < _B_GP - 1)
            def _():
                for cp in gathers(ci + _B_NB, b):
                    cp.start()

        return 0

    lax.fori_loop(0, _B_GP, grp, 0)
    for b in range(_B_NB):
        row0 = base + (_B_CH - _B_NB + b) * _B_CR
        pltpu.make_async_copy(out_v.at[b], out_hbm.at[pl.ds(row0, _B_CR)],
                              sos[b]).wait()


@functools.lru_cache(maxsize=None)
def _sc_bond_update_kernel():
    return pl.kernel(
        _sc_bond_update_body,
        mesh=_sc_mesh(),
        out_type=jax.ShapeDtypeStruct((E1P, H), jnp.float32),
        scratch_types=[
            pltpu.VMEM((_B_RW,), jnp.int32),
            pltpu.VMEM((_B_RW,), jnp.int32),
            pltpu.VMEM((_B_NB, _B_CR, H), jnp.float32),
            pltpu.VMEM((_B_NB, _B_CR, H), jnp.float32),
            pltpu.VMEM((_B_NB, _B_CR, H), jnp.float32),
            pltpu.SemaphoreType.DMA,
            pltpu.SemaphoreType.DMA,
            pltpu.SemaphoreType.DMA,
            pltpu.SemaphoreType.DMA,
            pltpu.SemaphoreType.DMA,
            pltpu.SemaphoreType.DMA,
            pltpu.SemaphoreType.DMA,
            pltpu.SemaphoreType.DMA,
            pltpu.SemaphoreType.DMA,
        ],
    )


def _sc_bond_update(amsg, msg, b2a_p, b2revb_p):
    return _sc_bond_update_kernel()(amsg, msg, b2a_p, b2revb_p)


_BM = 1024


def _dot_t(x, w):
    return lax.dot_general(x, w, (((1,), (1,)), ((), ())),
                           preferred_element_type=jnp.float32)


def _mm_inp_body(x_ref, w_ref, inp_ref, msg_ref):
    o = _dot_t(x_ref[...], w_ref[...])
    inp_ref[...] = o
    msg_ref[...] = jnp.maximum(o, 0.0)


def _mm_inp(fb, wi):
    n = fb.shape[0]
    return pl.pallas_call(
        _mm_inp_body,
        grid=(n // _BM,),
        in_specs=[
            pl.BlockSpec((_BM, BF), lambda i: (i, 0)),
            pl.BlockSpec((H, BF), lambda i: (0, 0)),
        ],
        out_specs=[
            pl.BlockSpec((_BM, H), lambda i: (i, 0)),
            pl.BlockSpec((_BM, H), lambda i: (i, 0)),
        ],
        out_shape=[
            jax.ShapeDtypeStruct((n, H), jnp.float32),
            jax.ShapeDtypeStruct((n, H), jnp.float32),
        ],
    )(fb, wi)


def _mm_layer_body(t_ref, inp_ref, w_ref, msg_ref):
    msg_ref[...] = jnp.maximum(inp_ref[...] + _dot_t(t_ref[...], w_ref[...]), 0.0)


def _mm_layer(t, inp, wh):
    n = t.shape[0]
    return pl.pallas_call(
        _mm_layer_body,
        grid=(n // _BM,),
        in_specs=[
            pl.BlockSpec((_BM, H), lambda i: (i, 0)),
            pl.BlockSpec((_BM, H), lambda i: (i, 0)),
            pl.BlockSpec((H, H), lambda i: (0, 0)),
        ],
        out_specs=pl.BlockSpec((_BM, H), lambda i: (i, 0)),
        out_shape=jax.ShapeDtypeStruct((n, H), jnp.float32),
    )(t, inp, wh)


_VM = 512


def _mm_val_body(fa_ref, am_ref, wa_ref, wh_ref, o_ref):
    o_ref[...] = _dot_t(fa_ref[...], wa_ref[...]) + _dot_t(am_ref[...], wh_ref[...])


def _mm_val(fa, am, wa, wh):
    n = fa.shape[0]
    return pl.pallas_call(
        _mm_val_body,
        grid=(n // _VM,),
        in_specs=[
            pl.BlockSpec((_VM, AF), lambda i: (i, 0)),
            pl.BlockSpec((_VM, H), lambda i: (i, 0)),
            pl.BlockSpec((H, AF), lambda i: (0, 0)),
            pl.BlockSpec((H, H), lambda i: (0, 0)),
        ],
        out_specs=pl.BlockSpec((_VM, H), lambda i: (i, 0)),
        out_shape=jax.ShapeDtypeStruct((n, H), jnp.float32),
    )(fa, am, wa, wh)


_AB = 8


def _attn_body(attn_ref, mval_ref, sval_ref, madd_ref, sadd_ref):
    a = attn_ref[...]
    mval = mval_ref[...]
    sval = sval_ref[...]

    s1 = jax.nn.softmax(a, axis=1)
    madd_ref[...] = lax.dot_general(
        s1, sval, (((1,), (1,)), ((0,), (0,))),
        preferred_element_type=jnp.float32)

    s2 = jax.nn.softmax(a, axis=2)
    sadd_ref[...] = lax.dot_general(
        s2, mval, (((2,), (1,)), ((0,), (0,))),
        preferred_element_type=jnp.float32)


def _attn(attn, mval, sval):
    return pl.pallas_call(
        _attn_body,
        grid=(B // _AB,),
        in_specs=[
            pl.BlockSpec((_AB, A, A), lambda i: (i, 0, 0)),
            pl.BlockSpec((_AB, A, H), lambda i: (i, 0, 0)),
            pl.BlockSpec((_AB, A, H), lambda i: (i, 0, 0)),
        ],
        out_specs=[
            pl.BlockSpec((_AB, A, H), lambda i: (i, 0, 0)),
            pl.BlockSpec((_AB, A, H), lambda i: (i, 0, 0)),
        ],
        out_shape=[
            jax.ShapeDtypeStruct((B, A, H), jnp.float32),
            jax.ShapeDtypeStruct((B, A, H), jnp.float32),
        ],
    )(attn, mval, sval)


def _mm_out_body(fa_ref, am_ref, add_ref, wa_ref, wh_ref, b_ref, o_ref):
    hid = _dot_t(fa_ref[...], wa_ref[...]) + _dot_t(am_ref[...], wh_ref[...])
    hid = jnp.maximum(hid + b_ref[...] + add_ref[...], 0.0)
    o_ref[...] = jnp.mean(hid.reshape(_VM // A, A, H), axis=1)


def _mm_out(fa, am, add, wa, wh, bias):
    n = fa.shape[0]
    return pl.pallas_call(
        _mm_out_body,
        grid=(n // _VM,),
        in_specs=[
            pl.BlockSpec((_VM, AF), lambda i: (i, 0)),
            pl.BlockSpec((_VM, H), lambda i: (i, 0)),
            pl.BlockSpec((_VM, H), lambda i: (i, 0)),
            pl.BlockSpec((H, AF), lambda i: (0, 0)),
            pl.BlockSpec((H, H), lambda i: (0, 0)),
            pl.BlockSpec((1, H), lambda i: (0, 0)),
        ],
        out_specs=pl.BlockSpec((_VM // A, H), lambda i: (i, 0)),
        out_shape=jax.ShapeDtypeStruct((n // A, H), jnp.float32),
    )(fa, am, add, wa, wh, bias)


def _pad_rows(x, n):
    return jnp.pad(x, ((0, n - x.shape[0]),) + ((0, 0),) * (x.ndim - 1))


def _mp_messages(fb, a2b, b2a, b2revb, wi, wh):
    fbp = _pad_rows(fb, E1P)
    a2b_f = _pad_rows(a2b.astype(jnp.int32).reshape(-1), N1P * MAXNB)
    b2a_p = _pad_rows(b2a.astype(jnp.int32), E1P)
    b2revb_p = _pad_rows(b2revb.astype(jnp.int32), E1P)

    inp, msg = _mm_inp(fbp, wi)
    for _ in range(DEPTH - 1):
        amsg = _sc_gather_sum(msg, a2b_f)
        t = _sc_bond_update(amsg, msg, b2a_p, b2revb_p)
        msg = _mm_layer(t, inp, wh)
    amsg = _sc_gather_sum(msg, a2b_f)
    return amsg[1:1 + B * A]


def kernel(mol_f_atoms, mol_f_bonds, struct_f_atoms, struct_f_bonds, attn_coefs,
           mol_a2b, mol_b2a, mol_b2revb, struct_a2b, struct_b2a, struct_b2revb,
           W_i1, W_i2, W_h1, W_h2, W_o1_w, W_o1_b, W_o2_w, W_o2_b, W_v1, W_v2):
    mol_am = _mp_messages(mol_f_bonds, mol_a2b, mol_b2a, mol_b2revb, W_i1, W_h1)
    struct_am = _mp_messages(struct_f_bonds, struct_a2b, struct_b2a,
                             struct_b2revb, W_i2, W_h2)

    mol_fa = mol_f_atoms[1:1 + B * A]
    struct_fa = struct_f_atoms[1:1 + B * A]

    mol_val = _mm_val(mol_fa, mol_am, W_v1[:, :AF], W_v1[:, AF:])
    struct_val = _mm_val(struct_fa, struct_am, W_v2[:, :AF], W_v2[:, AF:])

    mol_add, struct_add = _attn(attn_coefs,
                                mol_val.reshape(B, A, H),
                                struct_val.reshape(B, A, H))

    mol_vecs = _mm_out(mol_fa, mol_am, mol_add.reshape(B * A, H),
                       W_o1_w[:, :AF], W_o1_w[:, AF:], W_o1_b.reshape(1, H))
    struct_vecs = _mm_out(struct_fa, struct_am, struct_add.reshape(B * A, H),
                          W_o2_w[:, :AF], W_o2_w[:, AF:], W_o2_b.reshape(1, H))

    return jnp.concatenate([mol_vecs, struct_vecs], axis=1)


# speedup vs baseline: 1.0054x; 1.0054x over previous
import functools

import jax
import jax.numpy as jnp
from jax import lax
from jax.experimental import pallas as pl
from jax.experimental.pallas import tpu as pltpu
from jax.experimental.pallas import tpu_sc as plsc

B = 128
A = 64
N1 = B * A + 1
MAXNB = 16
E1 = B * A * MAXNB + 1
AF = 256
BF = 256
H = 512
DEPTH = 6

E1P = 132096
N1P = 8832

NC = 2
NS = 16
NW = NC * NS
L = 16

@functools.lru_cache(maxsize=None)
def _sc_mesh():
    return plsc.VectorSubcoreMesh(core_axis_name="c", subcore_axis_name="s",
                                  num_cores=NC, num_subcores=NS)


_A_RW = N1P // NW
_A_CR = 4
_A_NB = 3
_A_CH = _A_RW // _A_CR
_A_GP = _A_CH // _A_NB


def _sc_gather_sum_body(msg_hbm, a2b_hbm, out_hbm, idx_v, rows_v, acc_v,
                        sg0, sg1, sg2, so0, so1, so2):
    wid = lax.axis_index("s") * NC + lax.axis_index("c")
    base = wid * _A_RW
    sgs = [sg0, sg1, sg2]
    sos = [so0, so1, so2]

    pltpu.sync_copy(a2b_hbm.at[pl.ds(base * MAXNB, _A_RW * MAXNB)], idx_v)

    _HC = _A_CR * MAXNB // 2

    def gather(ci, b):
        i0 = ci * _A_CR * MAXNB
        return (pltpu.make_async_copy(
                    msg_hbm.at[idx_v.at[pl.ds(i0, _HC)]],
                    rows_v.at[b, pl.ds(0, _HC)], sgs[b]),
                pltpu.make_async_copy(
                    msg_hbm.at[idx_v.at[pl.ds(i0 + _HC, _HC)]],
                    rows_v.at[b, pl.ds(_HC, _HC)], sgs[b]))

    for b in range(_A_NB):
        for cp in gather(b, b):
            cp.start()

    def grp(g, _):
        for b in range(_A_NB):
            ci = g * _A_NB + b
            row0 = base + ci * _A_CR
            for cp in gather(ci, b):
                cp.wait()

            @pl.when(g > 0)
            def _():
                pltpu.make_async_copy(acc_v.at[b],
                                      out_hbm.at[pl.ds(row0, _A_CR)],
                                      sos[b]).wait()

            def row(r, _):
                def col(c, _):
                    cc = c * L
                    v = [rows_v[b, r * MAXNB + j, pl.ds(cc, L)]
                         for j in range(MAXNB)]
                    while len(v) > 1:
                        v = [v[i] + v[i + 1] for i in range(0, len(v), 2)]
                    acc_v[b, r, pl.ds(cc, L)] = v[0]
                    return 0

                lax.fori_loop(0, H // L, col, 0)
                return 0

            lax.fori_loop(0, _A_CR, row, 0)
            pltpu.make_async_copy(acc_v.at[b],
                                  out_hbm.at[pl.ds(row0, _A_CR)],
                                  sos[b]).start()

            @pl.when(g < _A_GP - 1)
            def _():
                for cp in gather(ci + _A_NB, b):
                    cp.start()

        return 0

    lax.fori_loop(0, _A_GP, grp, 0)
    for b in range(_A_NB):
        row0 = base + (_A_CH - _A_NB + b) * _A_CR
        pltpu.make_async_copy(acc_v.at[b], out_hbm.at[pl.ds(row0, _A_CR)],
                              sos[b]).wait()


@functools.lru_cache(maxsize=None)
def _sc_gather_sum_kernel():
    return pl.kernel(
        _sc_gather_sum_body,
        mesh=_sc_mesh(),
        out_type=jax.ShapeDtypeStruct((N1P, H), jnp.float32),
        scratch_types=[
            pltpu.VMEM((_A_RW * MAXNB,), jnp.int32),
            pltpu.VMEM((_A_NB, _A_CR * MAXNB, H), jnp.float32),
            pltpu.VMEM((_A_NB, _A_CR, H), jnp.float32),
            pltpu.SemaphoreType.DMA,
            pltpu.SemaphoreType.DMA,
            pltpu.SemaphoreType.DMA,
            pltpu.SemaphoreType.DMA,
            pltpu.SemaphoreType.DMA,
            pltpu.SemaphoreType.DMA,
        ],
    )


def _sc_gather_sum(msg, a2b_f):
    return _sc_gather_sum_kernel()(msg, a2b_f)


_B_RW = E1P // NW
_B_CR = 16
_B_NB = 3
_B_CH = _B_RW // _B_CR
_B_GP = _B_CH // _B_NB


def _sc_bond_update_body(amsg_hbm, msg_hbm, b2a_hbm, b2revb_hbm, out_hbm,
                         idxa_v, idxb_v, rows_a, rows_b, out_v,
                         sa0, sa1, sa2, sb0, sb1, sb2, so0, so1, so2):
    wid = lax.axis_index("s") * NC + lax.axis_index("c")
    base = wid * _B_RW
    sas = [sa0, sa1, sa2]
    sbs = [sb0, sb1, sb2]
    sos = [so0, so1, so2]

    pltpu.sync_copy(b2a_hbm.at[pl.ds(base, _B_RW)], idxa_v)
    pltpu.sync_copy(b2revb_hbm.at[pl.ds(base, _B_RW)], idxb_v)

    _HC = _B_CR // 2

    def gathers(ci, b):
        i0 = ci * _B_CR
        lo, hi = pl.ds(i0, _HC), pl.ds(i0 + _HC, _HC)
        return (pltpu.make_async_copy(amsg_hbm.at[idxa_v.at[lo]],
                                      rows_a.at[b, pl.ds(0, _HC)], sas[b]),
                pltpu.make_async_copy(amsg_hbm.at[idxa_v.at[hi]],
                                      rows_a.at[b, pl.ds(_HC, _HC)], sas[b]),
                pltpu.make_async_copy(msg_hbm.at[idxb_v.at[lo]],
                                      rows_b.at[b, pl.ds(0, _HC)], sbs[b]),
                pltpu.make_async_copy(msg_hbm.at[idxb_v.at[hi]],
                                      rows_b.at[b, pl.ds(_HC, _HC)], sbs[b]))

    for b in range(_B_NB):
        for cp in gathers(b, b):
            cp.start()

    def grp(g, _):
        for b in range(_B_NB):
            ci = g * _B_NB + b
            row0 = base + ci * _B_CR
            for cp in gathers(ci, b):
                cp.wait()

            @pl.when(g > 0)
            def _():
                pltpu.make_async_copy(out_v.at[b],
                                      out_hbm.at[pl.ds(row0, _B_CR)],
                                      sos[b]).wait()

            def row(r, _):
                def col(c, _):
                    cc = c * L
                    out_v[b, r, pl.ds(cc, L)] = (
                        rows_a[b, r, pl.ds(cc, L)] - rows_b[b, r, pl.ds(cc, L)]
                    )
                    return 0

                lax.fori_loop(0, H // L, col, 0)
                return 0

            lax.fori_loop(0, _B_CR, row, 0)
            pltpu.make_async_copy(out_v.at[b],
                                  out_hbm.at[pl.ds(row0, _B_CR)],
                                  sos[b]).start()

            @pl.when(g < _B_GP - 1)
            def _():
                for cp in gathers(ci + _B_NB, b):
                    cp.start()

        return 0

    lax.fori_loop(0, _B_GP, grp, 0)
    for b in range(_B_NB):
        row0 = base + (_B_CH - _B_NB + b) * _B_CR
        pltpu.make_async_copy(out_v.at[b], out_hbm.at[pl.ds(row0, _B_CR)],
                              sos[b]).wait()


@functools.lru_cache(maxsize=None)
def _sc_bond_update_kernel():
    return pl.kernel(
        _sc_bond_update_body,
        mesh=_sc_mesh(),
        out_type=jax.ShapeDtypeStruct((E1P, H), jnp.float32),
        scratch_types=[
            pltpu.VMEM((_B_RW,), jnp.int32),
            pltpu.VMEM((_B_RW,), jnp.int32),
            pltpu.VMEM((_B_NB, _B_CR, H), jnp.float32),
            pltpu.VMEM((_B_NB, _B_CR, H), jnp.float32),
            pltpu.VMEM((_B_NB, _B_CR, H), jnp.float32),
            pltpu.SemaphoreType.DMA,
            pltpu.SemaphoreType.DMA,
            pltpu.SemaphoreType.DMA,
            pltpu.SemaphoreType.DMA,
            pltpu.SemaphoreType.DMA,
            pltpu.SemaphoreType.DMA,
            pltpu.SemaphoreType.DMA,
            pltpu.SemaphoreType.DMA,
            pltpu.SemaphoreType.DMA,
        ],
    )


def _sc_bond_update(amsg, msg, b2a_p, b2revb_p):
    return _sc_bond_update_kernel()(amsg, msg, b2a_p, b2revb_p)


_BM = 1024


def _dot_t(x, w):
    return lax.dot_general(x, w, (((1,), (1,)), ((), ())),
                           preferred_element_type=jnp.float32)


def _mm_inp_body(x_ref, w_ref, inp_ref, msg_ref):
    o = _dot_t(x_ref[...], w_ref[...])
    inp_ref[...] = o
    msg_ref[...] = jnp.maximum(o, 0.0)


def _mm_inp(fb, wi):
    n = fb.shape[0]
    return pl.pallas_call(
        _mm_inp_body,
        grid=(n // _BM,),
        in_specs=[
            pl.BlockSpec((_BM, BF), lambda i: (i, 0)),
            pl.BlockSpec((H, BF), lambda i: (0, 0)),
        ],
        out_specs=[
            pl.BlockSpec((_BM, H), lambda i: (i, 0)),
            pl.BlockSpec((_BM, H), lambda i: (i, 0)),
        ],
        out_shape=[
            jax.ShapeDtypeStruct((n, H), jnp.float32),
            jax.ShapeDtypeStruct((n, H), jnp.float32),
        ],
    )(fb, wi)


def _mm_layer_body(t_ref, inp_ref, w_ref, msg_ref):
    msg_ref[...] = jnp.maximum(inp_ref[...] + _dot_t(t_ref[...], w_ref[...]), 0.0)


def _mm_layer(t, inp, wh):
    n = t.shape[0]
    return pl.pallas_call(
        _mm_layer_body,
        grid=(n // _BM,),
        in_specs=[
            pl.BlockSpec((_BM, H), lambda i: (i, 0)),
            pl.BlockSpec((_BM, H), lambda i: (i, 0)),
            pl.BlockSpec((H, H), lambda i: (0, 0)),
        ],
        out_specs=pl.BlockSpec((_BM, H), lambda i: (i, 0)),
        out_shape=jax.ShapeDtypeStruct((n, H), jnp.float32),
    )(t, inp, wh)


_VM = 512


def _mm_val_body(fa_ref, am_ref, wa_ref, wh_ref, o_ref):
    o_ref[...] = _dot_t(fa_ref[...], wa_ref[...]) + _dot_t(am_ref[...], wh_ref[...])


def _mm_val(fa, am, wa, wh):
    n = fa.shape[0]
    return pl.pallas_call(
        _mm_val_body,
        grid=(n // _VM,),
        in_specs=[
            pl.BlockSpec((_VM, AF), lambda i: (i, 0)),
            pl.BlockSpec((_VM, H), lambda i: (i, 0)),
            pl.BlockSpec((H, AF), lambda i: (0, 0)),
            pl.BlockSpec((H, H), lambda i: (0, 0)),
        ],
        out_specs=pl.BlockSpec((_VM, H), lambda i: (i, 0)),
        out_shape=jax.ShapeDtypeStruct((n, H), jnp.float32),
    )(fa, am, wa, wh)


_AB = 8


def _attn_body(attn_ref, mval_ref, sval_ref, madd_ref, sadd_ref):
    a = attn_ref[...]
    mval = mval_ref[...]
    sval = sval_ref[...]

    s1 = jax.nn.softmax(a, axis=1)
    madd_ref[...] = lax.dot_general(
        s1, sval, (((1,), (1,)), ((0,), (0,))),
        preferred_element_type=jnp.float32)

    s2 = jax.nn.softmax(a, axis=2)
    sadd_ref[...] = lax.dot_general(
        s2, mval, (((2,), (1,)), ((0,), (0,))),
        preferred_element_type=jnp.float32)


def _attn(attn, mval, sval):
    return pl.pallas_call(
        _attn_body,
        grid=(B // _AB,),
        in_specs=[
            pl.BlockSpec((_AB, A, A), lambda i: (i, 0, 0)),
            pl.BlockSpec((_AB, A, H), lambda i: (i, 0, 0)),
            pl.BlockSpec((_AB, A, H), lambda i: (i, 0, 0)),
        ],
        out_specs=[
            pl.BlockSpec((_AB, A, H), lambda i: (i, 0, 0)),
            pl.BlockSpec((_AB, A, H), lambda i: (i, 0, 0)),
        ],
        out_shape=[
            jax.ShapeDtypeStruct((B, A, H), jnp.float32),
            jax.ShapeDtypeStruct((B, A, H), jnp.float32),
        ],
    )(attn, mval, sval)


def _mm_out_body(fa_ref, am_ref, add_ref, wa_ref, wh_ref, b_ref, o_ref):
    hid = _dot_t(fa_ref[...], wa_ref[...]) + _dot_t(am_ref[...], wh_ref[...])
    hid = jnp.maximum(hid + b_ref[...] + add_ref[...], 0.0)
    o_ref[...] = jnp.mean(hid.reshape(_VM // A, A, H), axis=1)


def _mm_out(fa, am, add, wa, wh, bias):
    n = fa.shape[0]
    return pl.pallas_call(
        _mm_out_body,
        grid=(n // _VM,),
        in_specs=[
            pl.BlockSpec((_VM, AF), lambda i: (i, 0)),
            pl.BlockSpec((_VM, H), lambda i: (i, 0)),
            pl.BlockSpec((_VM, H), lambda i: (i, 0)),
            pl.BlockSpec((H, AF), lambda i: (0, 0)),
            pl.BlockSpec((H, H), lambda i: (0, 0)),
            pl.BlockSpec((1, H), lambda i: (0, 0)),
        ],
        out_specs=pl.BlockSpec((_VM // A, H), lambda i: (i, 0)),
        out_shape=jax.ShapeDtypeStruct((n // A, H), jnp.float32),
    )(fa, am, add, wa, wh, bias)


def _pad_rows(x, n):
    return jnp.pad(x, ((0, n - x.shape[0]),) + ((0, 0),) * (x.ndim - 1))


def _mp_messages(fb, a2b, b2a, b2revb, wi, wh):
    fbp = _pad_rows(fb, E1P)
    a2b_f = _pad_rows(a2b.astype(jnp.int32).reshape(-1), N1P * MAXNB)
    b2a_p = _pad_rows(b2a.astype(jnp.int32), E1P)
    b2revb_p = _pad_rows(b2revb.astype(jnp.int32), E1P)

    inp, msg = _mm_inp(fbp, wi)
    for _ in range(DEPTH - 1):
        amsg = _sc_gather_sum(msg, a2b_f)
        t = _sc_bond_update(amsg, msg, b2a_p, b2revb_p)
        msg = _mm_layer(t, inp, wh)
    amsg = _sc_gather_sum(msg, a2b_f)
    return amsg[1:1 + B * A]


def kernel(mol_f_atoms, mol_f_bonds, struct_f_atoms, struct_f_bonds, attn_coefs,
           mol_a2b, mol_b2a, mol_b2revb, struct_a2b, struct_b2a, struct_b2revb,
           W_i1, W_i2, W_h1, W_h2, W_o1_w, W_o1_b, W_o2_w, W_o2_b, W_v1, W_v2):
    mol_am = _mp_messages(mol_f_bonds, mol_a2b, mol_b2a, mol_b2revb, W_i1, W_h1)
    struct_am = _mp_messages(struct_f_bonds, struct_a2b, struct_b2a,
                             struct_b2revb, W_i2, W_h2)

    mol_fa = mol_f_atoms[1:1 + B * A]
    struct_fa = struct_f_atoms[1:1 + B * A]

    mol_val = _mm_val(mol_fa, mol_am, W_v1[:, :AF], W_v1[:, AF:])
    struct_val = _mm_val(struct_fa, struct_am, W_v2[:, :AF], W_v2[:, AF:])

    mol_add, struct_add = _attn(attn_coefs,
                                mol_val.reshape(B, A, H),
                                struct_val.reshape(B, A, H))

    mol_vecs = _mm_out(mol_fa, mol_am, mol_add.reshape(B * A, H),
                       W_o1_w[:, :AF], W_o1_w[:, AF:], W_o1_b.reshape(1, H))
    struct_vecs = _mm_out(struct_fa, struct_am, struct_add.reshape(B * A, H),
                          W_o2_w[:, :AF], W_o2_w[:, AF:], W_o2_b.reshape(1, H))

    return jnp.concatenate([mol_vecs, struct_vecs], axis=1)
